# baseline (speedup 1.0000x reference)
import functools

import jax
import jax.numpy as jnp
from jax import lax
from jax.experimental import pallas as pl
from jax.experimental.pallas import tpu as pltpu
from jax.experimental.pallas import tpu_sc as plsc

NUM_FEATS = 9
VOCAB = 100
HIDDEN = 32
N_ROWS = 100000

LANES = 16
NUM_WORKERS = 32
CHUNK = 800
GROUPS = CHUNK // LANES
NCHUNKS = N_ROWS // CHUNK
NROWS_TAB = NUM_FEATS * VOCAB
PLANE = 904
TAB_WORDS = LANES * PLANE


def _encoder_kernel(x_hbm, tab_hbm, out_hbm, tab_v, x_v, o_v):
    wid = lax.axis_index("s") * 2 + lax.axis_index("c")
    pltpu.sync_copy(tab_hbm, tab_v)

    iota = lax.iota(jnp.int32, LANES)
    row_stride = iota * NUM_FEATS
    plane_off = iota * PLANE

    def tree_sum(vals):
        while len(vals) > 1:
            vals = [a + b for a, b in zip(vals[::2], vals[1::2])] + (
                [vals[-1]] if len(vals) % 2 else [])
        return vals[0]

    def group_body(g):
        xbase = g * (LANES * NUM_FEATS)
        bvecs = []
        for i in range(NUM_FEATS):
            xi = plsc.load_gather(x_v, [xbase + i + row_stride])
            bvecs.append(xi + i * VOCAB)
        for r in range(LANES):
            rr = jnp.full((LANES,), r, jnp.int32)
            rows = []
            for i in range(NUM_FEATS):
                base = jnp.take(bvecs[i], rr, mode="fill")
                row = plsc.load_gather(tab_v, [base + plane_off])
                rows.append(plsc.bitcast(row, jnp.bfloat16))
            acc = tree_sum(rows)
            a, b = plsc.unpack(acc, format=plsc.PackFormat.INTERLEAVED)
            obase = (g * LANES + r) * HIDDEN
            o_v[pl.ds(obase, LANES)] = a
            o_v[pl.ds(obase + LANES, LANES)] = b

    def chunk_body(j, _):
        c = wid + j * NUM_WORKERS
        pltpu.sync_copy(
            x_hbm.at[pl.ds(c * (CHUNK * NUM_FEATS), CHUNK * NUM_FEATS)], x_v)
        plsc.parallel_loop(0, GROUPS)(group_body)
        pltpu.sync_copy(
            o_v, out_hbm.at[pl.ds(c * (CHUNK * HIDDEN), CHUNK * HIDDEN)])
        return 0

    nch = (NCHUNKS - wid + NUM_WORKERS - 1) // NUM_WORKERS
    lax.fori_loop(0, nch, chunk_body, 0)


@jax.jit
def _run(x_flat, tab_packed):
    mesh = plsc.VectorSubcoreMesh(core_axis_name="c", subcore_axis_name="s")
    f = functools.partial(
        pl.kernel,
        mesh=mesh,
        out_type=jax.ShapeDtypeStruct((N_ROWS * HIDDEN,), jnp.float32),
        compiler_params=pltpu.CompilerParams(needs_layout_passes=False),
        scratch_types=[
            pltpu.VMEM((TAB_WORDS,), jnp.int32),
            pltpu.VMEM((CHUNK * NUM_FEATS,), jnp.int32),
            pltpu.VMEM((CHUNK * HIDDEN,), jnp.float32),
        ],
    )(_encoder_kernel)
    return f(x_flat, tab_packed)


def kernel(x, tables):
    if x.ndim == 1:
        x = x[:, None]
    x_flat = x.astype(jnp.int32).reshape(-1)
    tab_bf = tables.astype(jnp.bfloat16).reshape(NROWS_TAB, HIDDEN)
    pairs = jnp.stack([tab_bf[:, :LANES], tab_bf[:, LANES:]], axis=-1)
    packed = lax.bitcast_convert_type(pairs, jnp.int32)
    planes = jnp.pad(packed.T, ((0, 0), (0, PLANE - NROWS_TAB)))
    tab_packed = planes.reshape(-1)
    out = _run(x_flat, tab_packed)
    return out.reshape(N_ROWS, HIDDEN)


# speedup vs baseline: 1.0472x; 1.0472x over previous
import functools

import jax
import jax.numpy as jnp
from jax import lax
from jax.experimental import pallas as pl
from jax.experimental.pallas import tpu as pltpu
from jax.experimental.pallas import tpu_sc as plsc

NUM_FEATS = 9
VOCAB = 100
HIDDEN = 32
N_ROWS = 100000

LANES = 16
NUM_WORKERS = 32
CHUNK = 800
GROUPS = CHUNK // LANES
NCHUNKS = N_ROWS // CHUNK
NROWS_TAB = NUM_FEATS * VOCAB
TAB_WORDS = NROWS_TAB * LANES


def _encoder_kernel(x_hbm, tab_hbm, out_hbm, tab_v, x_v, o_v):
    wid = lax.axis_index("s") * 2 + lax.axis_index("c")
    pltpu.sync_copy(tab_hbm, tab_v)

    iota = lax.iota(jnp.int32, LANES)
    row_stride = iota * NUM_FEATS

    def tree_sum(vals):
        while len(vals) > 1:
            vals = [a + b for a, b in zip(vals[::2], vals[1::2])] + (
                [vals[-1]] if len(vals) % 2 else [])
        return vals[0]

    def group_body(g):
        xbase = g * (LANES * NUM_FEATS)
        bvecs = []
        for i in range(NUM_FEATS):
            xi = plsc.load_gather(x_v, [xbase + i + row_stride])
            bvecs.append((xi + i * VOCAB) * LANES)
        for r in range(LANES):
            rows = []
            for i in range(NUM_FEATS):
                row = tab_v[pl.ds(bvecs[i][r], LANES)]
                rows.append(plsc.bitcast(row, jnp.bfloat16))
            acc = tree_sum(rows)
            a, b = plsc.unpack(acc, format=plsc.PackFormat.INTERLEAVED)
            obase = (g * LANES + r) * HIDDEN
            o_v[pl.ds(obase, LANES)] = a
            o_v[pl.ds(obase + LANES, LANES)] = b

    def chunk_body(j, _):
        c = wid + j * NUM_WORKERS
        pltpu.sync_copy(
            x_hbm.at[pl.ds(c * (CHUNK * NUM_FEATS), CHUNK * NUM_FEATS)], x_v)
        plsc.parallel_loop(0, GROUPS)(group_body)
        pltpu.sync_copy(
            o_v, out_hbm.at[pl.ds(c * (CHUNK * HIDDEN), CHUNK * HIDDEN)])
        return 0

    nch = (NCHUNKS - wid + NUM_WORKERS - 1) // NUM_WORKERS
    lax.fori_loop(0, nch, chunk_body, 0)


@jax.jit
def _run(x_flat, tab_packed):
    mesh = plsc.VectorSubcoreMesh(core_axis_name="c", subcore_axis_name="s")
    f = functools.partial(
        pl.kernel,
        mesh=mesh,
        out_type=jax.ShapeDtypeStruct((N_ROWS * HIDDEN,), jnp.float32),
        compiler_params=pltpu.CompilerParams(needs_layout_passes=False),
        scratch_types=[
            pltpu.VMEM((TAB_WORDS,), jnp.int32),
            pltpu.VMEM((CHUNK * NUM_FEATS,), jnp.int32),
            pltpu.VMEM((CHUNK * HIDDEN,), jnp.float32),
        ],
    )(_encoder_kernel)
    return f(x_flat, tab_packed)


def kernel(x, tables):
    if x.ndim == 1:
        x = x[:, None]
    x_flat = x.astype(jnp.int32).reshape(-1)
    tab_bf = tables.astype(jnp.bfloat16).reshape(NROWS_TAB, HIDDEN)
    pairs = jnp.stack([tab_bf[:, :LANES], tab_bf[:, LANES:]], axis=-1)
    packed = lax.bitcast_convert_type(pairs, jnp.int32)
    tab_packed = packed.reshape(-1)
    out = _run(x_flat, tab_packed)
    return out.reshape(N_ROWS, HIDDEN)


# speedup vs baseline: 1.0748x; 1.0264x over previous
import functools

import jax
import jax.numpy as jnp
from jax import lax
from jax.experimental import pallas as pl
from jax.experimental.pallas import tpu as pltpu
from jax.experimental.pallas import tpu_sc as plsc

NUM_FEATS = 9
VOCAB = 100
HIDDEN = 32
N_ROWS = 100000

LANES = 16
NUM_WORKERS = 32
CHUNK = 800
GROUPS = CHUNK // LANES
NCHUNKS = N_ROWS // CHUNK
NROWS_TAB = NUM_FEATS * VOCAB
TAB_WORDS = NROWS_TAB * LANES


def _encoder_kernel(x_hbm, tab_hbm, out_hbm, tab_v,
                    x_v0, x_v1, o_v0, o_v1, sx0, sx1, so0, so1):
    wid = lax.axis_index("s") * 2 + lax.axis_index("c")
    pltpu.sync_copy(tab_hbm, tab_v)

    iota = lax.iota(jnp.int32, LANES)
    row_stride = iota * NUM_FEATS

    def tree_sum(vals):
        while len(vals) > 1:
            vals = [a + b for a, b in zip(vals[::2], vals[1::2])] + (
                [vals[-1]] if len(vals) % 2 else [])
        return vals[0]

    def make_group_body(x_v, o_v):
        def group_body(g):
            xbase = g * (LANES * NUM_FEATS)
            bvecs = []
            for i in range(NUM_FEATS):
                xi = plsc.load_gather(x_v, [xbase + i + row_stride])
                bvecs.append((xi + i * VOCAB) * LANES)
            for r in range(LANES):
                rows = []
                for i in range(NUM_FEATS):
                    row = tab_v[pl.ds(bvecs[i][r], LANES)]
                    rows.append(plsc.bitcast(row, jnp.bfloat16))
                acc = tree_sum(rows)
                a, b = plsc.unpack(acc, format=plsc.PackFormat.INTERLEAVED)
                obase = (g * LANES + r) * HIDDEN
                o_v[pl.ds(obase, LANES)] = a
                o_v[pl.ds(obase + LANES, LANES)] = b
        return group_body

    def x_slice(c):
        return x_hbm.at[pl.ds(c * (CHUNK * NUM_FEATS), CHUNK * NUM_FEATS)]

    def o_slice(c):
        return out_hbm.at[pl.ds(c * (CHUNK * HIDDEN), CHUNK * HIDDEN)]

    nch = (NCHUNKS - wid + NUM_WORKERS - 1) // NUM_WORKERS

    pltpu.make_async_copy(x_slice(wid), x_v0, sx0).start()

    def pair_body(jj, _):
        j0 = 2 * jj
        j1 = j0 + 1
        c0 = wid + j0 * NUM_WORKERS
        c1 = wid + j1 * NUM_WORKERS
        pltpu.make_async_copy(x_slice(c0), x_v0, sx0).wait()

        @pl.when(j1 < nch)
        def _():
            pltpu.make_async_copy(x_slice(c1), x_v1, sx1).start()

        @pl.when(jj > 0)
        def _():
            pltpu.make_async_copy(o_v0, o_slice(c0), so0).wait()

        plsc.parallel_loop(0, GROUPS)(make_group_body(x_v0, o_v0))
        pltpu.make_async_copy(o_v0, o_slice(c0), so0).start()

        @pl.when(j1 < nch)
        def _():
            pltpu.make_async_copy(x_slice(c1), x_v1, sx1).wait()

            @pl.when(j0 + 2 < nch)
            def _():
                pltpu.make_async_copy(
                    x_slice(wid + (j0 + 2) * NUM_WORKERS), x_v0, sx0).start()

            @pl.when(jj > 0)
            def _():
                pltpu.make_async_copy(o_v1, o_slice(c1), so1).wait()

            plsc.parallel_loop(0, GROUPS)(make_group_body(x_v1, o_v1))
            pltpu.make_async_copy(o_v1, o_slice(c1), so1).start()
        return 0

    lax.fori_loop(0, (nch + 1) // 2, pair_body, 0)

    pltpu.make_async_copy(o_v0, o_slice(wid), so0).wait()

    @pl.when(nch >= 2)
    def _():
        pltpu.make_async_copy(o_v1, o_slice(wid), so1).wait()


@jax.jit
def _run(x_flat, tab_packed):
    mesh = plsc.VectorSubcoreMesh(core_axis_name="c", subcore_axis_name="s")
    f = functools.partial(
        pl.kernel,
        mesh=mesh,
        out_type=jax.ShapeDtypeStruct((N_ROWS * HIDDEN,), jnp.float32),
        compiler_params=pltpu.CompilerParams(needs_layout_passes=False),
        scratch_types=[
            pltpu.VMEM((TAB_WORDS,), jnp.int32),
            pltpu.VMEM((CHUNK * NUM_FEATS,), jnp.int32),
            pltpu.VMEM((CHUNK * NUM_FEATS,), jnp.int32),
            pltpu.VMEM((CHUNK * HIDDEN,), jnp.float32),
            pltpu.VMEM((CHUNK * HIDDEN,), jnp.float32),
            pltpu.SemaphoreType.DMA,
            pltpu.SemaphoreType.DMA,
            pltpu.SemaphoreType.DMA,
            pltpu.SemaphoreType.DMA,
        ],
    )(_encoder_kernel)
    return f(x_flat, tab_packed)


def kernel(x, tables):
    if x.ndim == 1:
        x = x[:, None]
    x_flat = x.astype(jnp.int32).reshape(-1)
    tab_bf = tables.astype(jnp.bfloat16).reshape(NROWS_TAB, HIDDEN)
    pairs = jnp.stack([tab_bf[:, :LANES], tab_bf[:, LANES:]], axis=-1)
    packed = lax.bitcast_convert_type(pairs, jnp.int32)
    tab_packed = packed.reshape(-1)
    out = _run(x_flat, tab_packed)
    return out.reshape(N_ROWS, HIDDEN)
